# baseline (speedup 1.0000x reference)
import functools

import jax
import jax.numpy as jnp
from jax.experimental import pallas as pl

B = 16
T_MEL = 1000
T_SRC = 128
N_MEL = 80
K_EMB = 32
D_EMB = 256
BLANK_LOGPROB = -1.0
BIN_ENABLE = 1000
BIN_WARMUP = 2000
NEG = -1e30

_INTERPRET = False


def _lse2(a, b):
    m = jnp.maximum(a, b)
    return m + jnp.log1p(jnp.exp(-jnp.abs(a - b)))


def _lse3(a, b, c):
    m = jnp.maximum(jnp.maximum(a, b), c)
    return m + jnp.log(jnp.exp(a - m) + jnp.exp(b - m) + jnp.exp(c - m))


_CH = 20
_NC = T_MEL // _CH
_NG = 2
_G = B // _NG


def _ctc_kernel(x_ref, k128_ref, src_ref, mel_ref, out_ref):
    K = src_ref[...]
    Tb = mel_ref[...]
    K128 = k128_ref[...]
    jfull = jax.lax.broadcasted_iota(jnp.int32, (_CH * B, T_SRC), 1)
    labmask = jfull < K128

    def chunk_logprobs(c):
        xblk = x_ref[c]
        xm = jnp.where(labmask, xblk, NEG)
        m = jnp.maximum(jnp.max(xm, axis=1, keepdims=True), BLANK_LOGPROB)
        z = m + jnp.log(jnp.sum(jnp.exp(xm - m), axis=1, keepdims=True)
                        + jnp.exp(BLANK_LOGPROB - m))
        lp_lab = jnp.where(labmask, xblk - z, NEG)
        lp_b = BLANK_LOGPROB - z
        return lp_lab, lp_b

    neg_col = jnp.full((_G, 1), NEG, dtype=jnp.float32)

    def step(carry, lp_lab_k, lp_b_k, keep):
        even, odd = carry
        ood = jnp.concatenate([neg_col, odd], axis=1)
        new_even = _lse2(even, ood) + lp_b_k
        new_odd = _lse3(odd, even[:, :T_SRC], ood[:, :T_SRC]) + lp_lab_k
        return jnp.where(keep, new_even, even), jnp.where(keep, new_odd, odd)

    def nostep(carry, lp_lab_k, lp_b_k):
        even, odd = carry
        ood = jnp.concatenate([neg_col, odd], axis=1)
        new_even = _lse2(even, ood) + lp_b_k
        new_odd = _lse3(odd, even[:, :T_SRC], ood[:, :T_SRC]) + lp_lab_k
        return new_even, new_odd

    j129g = jax.lax.broadcasted_iota(jnp.int32, (_G, T_SRC + 1), 1)
    j128g = jax.lax.broadcasted_iota(jnp.int32, (_G, T_SRC), 1)
    Tb_g = [Tb[g * _G:(g + 1) * _G] for g in range(_NG)]

    lp_lab, lp_b = chunk_logprobs(0)

    def init_group(g):
        lo = g * _G
        lab0 = lp_lab[lo:lo + _G]
        b0 = lp_b[lo:lo + _G]
        even = jnp.where(j129g == 0, b0, NEG)
        odd = jnp.where(j128g == 0, lab0[:, 0:1], NEG)
        return even, odd

    cars = tuple(init_group(g) for g in range(_NG))

    def run_chunk(c_base, lp_lab, lp_b, cars, k_start, freeze):
        cars = list(cars)
        for k in range(k_start, _CH):
            for g in range(_NG):
                r = k * B + g * _G
                if freeze:
                    keep = (c_base + k) < Tb_g[g]
                    cars[g] = step(cars[g], lp_lab[r:r + _G],
                                   lp_b[r:r + _G], keep)
                else:
                    cars[g] = nostep(cars[g], lp_lab[r:r + _G],
                                     lp_b[r:r + _G])
        return tuple(cars)

    cars = run_chunk(0, lp_lab, lp_b, cars, 1, True)

    tmax = jnp.max(Tb)
    tmin = jnp.min(Tb)
    cmax = (tmax + _CH - 1) // _CH
    cnf = jnp.maximum((tmin - _CH) // _CH + 1, 1)

    def body_nf(c, carry):
        cars, lp_lab, lp_b = carry
        nlab, nb = chunk_logprobs(jnp.minimum(c + 1, _NC - 1))
        cars = run_chunk(c * _CH, lp_lab, lp_b, cars, 0, False)
        return cars, nlab, nb

    def body_fr(c, carry):
        cars, lp_lab, lp_b = carry
        nlab, nb = chunk_logprobs(jnp.minimum(c + 1, _NC - 1))
        cars = run_chunk(c * _CH, lp_lab, lp_b, cars, 0, True)
        return cars, nlab, nb

    carry = (cars,) + chunk_logprobs(1)
    carry = jax.lax.fori_loop(1, cnf, body_nf, carry)
    carry = jax.lax.fori_loop(cnf, cmax, body_fr, carry)
    cars = carry[0]

    def finish(car, Kg):
        even, odd = car
        ev = jnp.sum(jnp.where(j129g == Kg, even, 0.0), axis=1)
        od = jnp.sum(jnp.where(j128g == Kg - 1, odd, 0.0), axis=1)
        ll = _lse2(ev, od)
        return jnp.sum(-ll / Kg[:, 0].astype(jnp.float32))

    total = sum(finish(cars[g], K[g * _G:(g + 1) * _G])
                for g in range(_NG)) / B
    lane = jax.lax.broadcasted_iota(jnp.int32, (1, 128), 1)
    out_ref[...] = jnp.where(lane == 0, total, 0.0)


def _ctc_forward_sum(attn_logprob, src_lens, mel_lens):
    xT = jnp.transpose(attn_logprob[:, 0], (1, 0, 2)).reshape(_NC, _CH * B, T_SRC)
    k128 = jnp.tile(src_lens, _CH).reshape(_CH * B, 1)
    out = pl.pallas_call(
        _ctc_kernel,
        out_shape=jax.ShapeDtypeStruct((1, 128), jnp.float32),
        interpret=_INTERPRET,
    )(xT, k128, src_lens.reshape(B, 1), mel_lens.reshape(B, 1))
    return out[0, 0]


_NT = 5
_TC = T_MEL // _NT


def _dense_kernel(mel_p_ref, post_ref, mel_t_ref, soft_ref, hard_ref,
                  mel_len_ref, a_ref, b_ref, out_ref):
    i = pl.program_id(0)

    @pl.when(i == 0)
    def _init():
        out_ref[...] = jnp.zeros_like(out_ref)

    Tb = mel_len_ref[...]
    t_loc = jax.lax.broadcasted_iota(jnp.int32, (B, _TC), 1)
    mask = ((i * _TC + t_loc) < Tb).astype(jnp.float32)

    d1 = jnp.sum(jnp.abs(mel_p_ref[...] - mel_t_ref[...]), axis=2)
    d2 = jnp.sum(jnp.abs(post_ref[...] - mel_t_ref[...]), axis=2)
    soft = soft_ref[...]
    hard = hard_ref[...]

    lane = jax.lax.broadcasted_iota(jnp.int32, (1, 8), 1)
    row = jnp.where(lane == 0, jnp.sum(d1 * mask), 0.0)
    row += jnp.where(lane == 1, jnp.sum(d2 * mask), 0.0)
    row += jnp.where(lane == 2,
                     jnp.sum(jnp.log(jnp.clip(soft, 1e-12, None)) * hard), 0.0)
    row += jnp.where(lane == 3, jnp.sum(hard), 0.0)
    out_ref[...] += row

    @pl.when(i == 0)
    def _cos():
        eye = jnp.eye(K_EMB, dtype=jnp.float32)
        crow = jnp.zeros((1, 8), jnp.float32)
        for p in range(4):
            a = a_ref[p]
            b = b_ref[p]
            acc = jnp.float32(0.0)
            for bb in range(B):
                ab = a[bb]
                bbm = b[bb]
                an = jnp.sqrt(jnp.sum(ab * ab, axis=1))
                bn = jnp.sqrt(jnp.sum(bbm * bbm, axis=1))
                dots = jax.lax.dot_general(
                    ab, bbm, (((1,), (1,)), ((), ())),
                    preferred_element_type=jnp.float32)
                cos = dots / jnp.maximum(an[:, None] * bn[None, :], 1e-8)
                acc += jnp.sum((cos - eye) ** 2)
            crow += jnp.where(lane == 4 + p, acc, 0.0)
        out_ref[...] += crow


def _dense_losses(mel_p, post, mel_t, soft, hard, mel_lens, a_stack, b_stack):
    out = pl.pallas_call(
        _dense_kernel,
        grid=(_NT,),
        in_specs=[
            pl.BlockSpec((B, _TC, N_MEL), lambda i: (0, i, 0)),
            pl.BlockSpec((B, _TC, N_MEL), lambda i: (0, i, 0)),
            pl.BlockSpec((B, _TC, N_MEL), lambda i: (0, i, 0)),
            pl.BlockSpec((B, _TC, T_SRC), lambda i: (0, i, 0)),
            pl.BlockSpec((B, _TC, T_SRC), lambda i: (0, i, 0)),
            pl.BlockSpec((B, 1), lambda i: (0, 0)),
            pl.BlockSpec((4, B, K_EMB, D_EMB), lambda i: (0, 0, 0, 0)),
            pl.BlockSpec((4, B, K_EMB, D_EMB), lambda i: (0, 0, 0, 0)),
        ],
        out_specs=pl.BlockSpec((1, 8), lambda i: (0, 0)),
        out_shape=jax.ShapeDtypeStruct((1, 8), jnp.float32),
        interpret=_INTERPRET,
    )(mel_p, post, mel_t, soft, hard, mel_lens, a_stack, b_stack)
    return out[0]


@functools.partial(jax.jit, static_argnames=())
def kernel(mel_predictions, postnet_mel_predictions, mel_targets,
           pitch_predictions, pitch_targets, energy_predictions,
           energy_targets, attn_soft, attn_hard, attn_logprob, src_lens,
           mel_lens, style_emb, cross_style_memory_enhancement_emb,
           linguistic_emb, cross_linguistic_memory_enhancement_emb,
           intra_style_memory_enhancement_emb,
           intra_linguistic_memory_enhancement_emb, step):
    a_stack = jnp.stack([linguistic_emb, style_emb, style_emb, linguistic_emb])
    b_stack = jnp.stack([cross_style_memory_enhancement_emb,
                         cross_linguistic_memory_enhancement_emb,
                         intra_style_memory_enhancement_emb,
                         intra_linguistic_memory_enhancement_emb])
    sums = _dense_losses(mel_predictions, postnet_mel_predictions,
                         mel_targets, attn_soft[:, 0], attn_hard[:, 0],
                         mel_lens.reshape(B, 1), a_stack, b_stack)
    ctc = _ctc_forward_sum(attn_logprob, src_lens, mel_lens)

    denom = jnp.sum(mel_lens).astype(jnp.float32) * N_MEL
    mel_loss = sums[0] / denom
    postnet_mel_loss = sums[1] / denom
    bw = jnp.where(step < BIN_ENABLE, 0.0,
                   jnp.minimum((step - BIN_ENABLE) / BIN_WARMUP, 1.0))
    bin_loss = -(sums[2] / sums[3]) * bw
    cos_losses = jnp.sum(sums[4:8]) / (B * K_EMB * K_EMB)
    return mel_loss + postnet_mel_loss + ctc + bin_loss + cos_losses


# speedup vs baseline: 1.0015x; 1.0015x over previous
import functools

import jax
import jax.numpy as jnp
from jax.experimental import pallas as pl

B = 16
T_MEL = 1000
T_SRC = 128
N_MEL = 80
K_EMB = 32
D_EMB = 256
BLANK_LOGPROB = -1.0
BIN_ENABLE = 1000
BIN_WARMUP = 2000
NEG = -1e30

def _lse2(a, b):
    m = jnp.maximum(a, b)
    return m + jnp.log1p(jnp.exp(-jnp.abs(a - b)))


def _lse3(a, b, c):
    m = jnp.maximum(jnp.maximum(a, b), c)
    return m + jnp.log(jnp.exp(a - m) + jnp.exp(b - m) + jnp.exp(c - m))


_CH = 20
_NC = T_MEL // _CH
_NG = 2
_G = B // _NG


def _ctc_kernel(x_ref, k128_ref, src_ref, mel_ref, out_ref):
    K = src_ref[...]
    Tb = mel_ref[...]
    K128 = k128_ref[...]
    jfull = jax.lax.broadcasted_iota(jnp.int32, (_CH * B, T_SRC), 1)
    labmask = jfull < K128

    def chunk_logprobs(c):
        xblk = x_ref[c]
        xm = jnp.where(labmask, xblk, NEG)
        m = jnp.maximum(jnp.max(xm, axis=1, keepdims=True), BLANK_LOGPROB)
        z = m + jnp.log(jnp.sum(jnp.exp(xm - m), axis=1, keepdims=True)
                        + jnp.exp(BLANK_LOGPROB - m))
        lp_lab = jnp.where(labmask, xblk - z, NEG)
        lp_b = BLANK_LOGPROB - z
        return lp_lab, lp_b

    neg_col = jnp.full((_G, 1), NEG, dtype=jnp.float32)

    def step(carry, lp_lab_k, lp_b_k, keep):
        even, odd = carry
        ood = jnp.concatenate([neg_col, odd], axis=1)
        new_even = _lse2(even, ood) + lp_b_k
        new_odd = _lse3(odd, even[:, :T_SRC], ood[:, :T_SRC]) + lp_lab_k
        return jnp.where(keep, new_even, even), jnp.where(keep, new_odd, odd)

    def nostep(carry, lp_lab_k, lp_b_k):
        even, odd = carry
        ood = jnp.concatenate([neg_col, odd], axis=1)
        new_even = _lse2(even, ood) + lp_b_k
        new_odd = _lse3(odd, even[:, :T_SRC], ood[:, :T_SRC]) + lp_lab_k
        return new_even, new_odd

    j129g = jax.lax.broadcasted_iota(jnp.int32, (_G, T_SRC + 1), 1)
    j128g = jax.lax.broadcasted_iota(jnp.int32, (_G, T_SRC), 1)
    Tb_g = [Tb[g * _G:(g + 1) * _G] for g in range(_NG)]

    lp_lab, lp_b = chunk_logprobs(0)

    def init_group(g):
        lo = g * _G
        lab0 = lp_lab[lo:lo + _G]
        b0 = lp_b[lo:lo + _G]
        even = jnp.where(j129g == 0, b0, NEG)
        odd = jnp.where(j128g == 0, lab0[:, 0:1], NEG)
        return even, odd

    cars = tuple(init_group(g) for g in range(_NG))

    def run_chunk(c_base, lp_lab, lp_b, cars, k_start, freeze):
        cars = list(cars)
        for k in range(k_start, _CH):
            for g in range(_NG):
                r = k * B + g * _G
                if freeze:
                    keep = (c_base + k) < Tb_g[g]
                    cars[g] = step(cars[g], lp_lab[r:r + _G],
                                   lp_b[r:r + _G], keep)
                else:
                    cars[g] = nostep(cars[g], lp_lab[r:r + _G],
                                     lp_b[r:r + _G])
        return tuple(cars)

    cars = run_chunk(0, lp_lab, lp_b, cars, 1, True)

    tmax = jnp.max(Tb)
    tmin = jnp.min(Tb)
    cmax = (tmax + _CH - 1) // _CH
    cnf = jnp.maximum((tmin - _CH) // _CH + 1, 1)

    def body_nf(c, carry):
        cars, lp_lab, lp_b = carry
        nlab, nb = chunk_logprobs(jnp.minimum(c + 1, _NC - 1))
        cars = run_chunk(c * _CH, lp_lab, lp_b, cars, 0, False)
        return cars, nlab, nb

    def body_fr(c, carry):
        cars, lp_lab, lp_b = carry
        nlab, nb = chunk_logprobs(jnp.minimum(c + 1, _NC - 1))
        cars = run_chunk(c * _CH, lp_lab, lp_b, cars, 0, True)
        return cars, nlab, nb

    carry = (cars,) + chunk_logprobs(1)
    carry = jax.lax.fori_loop(1, cnf, body_nf, carry)
    carry = jax.lax.fori_loop(cnf, cmax, body_fr, carry)
    cars = carry[0]

    def finish(car, Kg):
        even, odd = car
        ev = jnp.sum(jnp.where(j129g == Kg, even, 0.0), axis=1)
        od = jnp.sum(jnp.where(j128g == Kg - 1, odd, 0.0), axis=1)
        ll = _lse2(ev, od)
        return jnp.sum(-ll / Kg[:, 0].astype(jnp.float32))

    total = sum(finish(cars[g], K[g * _G:(g + 1) * _G])
                for g in range(_NG)) / B
    lane = jax.lax.broadcasted_iota(jnp.int32, (1, 128), 1)
    out_ref[...] = jnp.where(lane == 0, total, 0.0)


def _ctc_forward_sum(attn_logprob, src_lens, mel_lens):
    xT = jnp.transpose(attn_logprob[:, 0], (1, 0, 2)).reshape(_NC, _CH * B, T_SRC)
    k128 = jnp.tile(src_lens, _CH).reshape(_CH * B, 1)
    out = pl.pallas_call(
        _ctc_kernel,
        out_shape=jax.ShapeDtypeStruct((1, 128), jnp.float32),
    )(xT, k128, src_lens.reshape(B, 1), mel_lens.reshape(B, 1))
    return out[0, 0]


_NT = 5
_TC = T_MEL // _NT


def _dense_kernel(mel_p_ref, post_ref, mel_t_ref, soft_ref, hard_ref,
                  mel_len_ref, a_ref, b_ref, out_ref):
    i = pl.program_id(0)

    @pl.when(i == 0)
    def _init():
        out_ref[...] = jnp.zeros_like(out_ref)

    Tb = mel_len_ref[...]
    t_loc = jax.lax.broadcasted_iota(jnp.int32, (B, _TC), 1)
    mask = ((i * _TC + t_loc) < Tb).astype(jnp.float32)

    d1 = jnp.sum(jnp.abs(mel_p_ref[...] - mel_t_ref[...]), axis=2)
    d2 = jnp.sum(jnp.abs(post_ref[...] - mel_t_ref[...]), axis=2)
    soft = soft_ref[...]
    hard = hard_ref[...]

    lane = jax.lax.broadcasted_iota(jnp.int32, (1, 8), 1)
    row = jnp.where(lane == 0, jnp.sum(d1 * mask), 0.0)
    row += jnp.where(lane == 1, jnp.sum(d2 * mask), 0.0)
    row += jnp.where(lane == 2,
                     jnp.sum(jnp.log(jnp.clip(soft, 1e-12, None)) * hard), 0.0)
    row += jnp.where(lane == 3, jnp.sum(hard), 0.0)
    out_ref[...] += row

    @pl.when(i == 0)
    def _cos():
        eye = jnp.eye(K_EMB, dtype=jnp.float32)
        crow = jnp.zeros((1, 8), jnp.float32)
        for p in range(4):
            a = a_ref[p]
            b = b_ref[p]
            acc = jnp.float32(0.0)
            for bb in range(B):
                ab = a[bb]
                bbm = b[bb]
                an = jnp.sqrt(jnp.sum(ab * ab, axis=1))
                bn = jnp.sqrt(jnp.sum(bbm * bbm, axis=1))
                dots = jax.lax.dot_general(
                    ab, bbm, (((1,), (1,)), ((), ())),
                    preferred_element_type=jnp.float32)
                cos = dots / jnp.maximum(an[:, None] * bn[None, :], 1e-8)
                acc += jnp.sum((cos - eye) ** 2)
            crow += jnp.where(lane == 4 + p, acc, 0.0)
        out_ref[...] += crow


def _dense_losses(mel_p, post, mel_t, soft, hard, mel_lens, a_stack, b_stack):
    out = pl.pallas_call(
        _dense_kernel,
        grid=(_NT,),
        in_specs=[
            pl.BlockSpec((B, _TC, N_MEL), lambda i: (0, i, 0)),
            pl.BlockSpec((B, _TC, N_MEL), lambda i: (0, i, 0)),
            pl.BlockSpec((B, _TC, N_MEL), lambda i: (0, i, 0)),
            pl.BlockSpec((B, _TC, T_SRC), lambda i: (0, i, 0)),
            pl.BlockSpec((B, _TC, T_SRC), lambda i: (0, i, 0)),
            pl.BlockSpec((B, 1), lambda i: (0, 0)),
            pl.BlockSpec((4, B, K_EMB, D_EMB), lambda i: (0, 0, 0, 0)),
            pl.BlockSpec((4, B, K_EMB, D_EMB), lambda i: (0, 0, 0, 0)),
        ],
        out_specs=pl.BlockSpec((1, 8), lambda i: (0, 0)),
        out_shape=jax.ShapeDtypeStruct((1, 8), jnp.float32),
    )(mel_p, post, mel_t, soft, hard, mel_lens, a_stack, b_stack)
    return out[0]


@functools.partial(jax.jit, static_argnames=())
def kernel(mel_predictions, postnet_mel_predictions, mel_targets,
           pitch_predictions, pitch_targets, energy_predictions,
           energy_targets, attn_soft, attn_hard, attn_logprob, src_lens,
           mel_lens, style_emb, cross_style_memory_enhancement_emb,
           linguistic_emb, cross_linguistic_memory_enhancement_emb,
           intra_style_memory_enhancement_emb,
           intra_linguistic_memory_enhancement_emb, step):
    a_stack = jnp.stack([linguistic_emb, style_emb, style_emb, linguistic_emb])
    b_stack = jnp.stack([cross_style_memory_enhancement_emb,
                         cross_linguistic_memory_enhancement_emb,
                         intra_style_memory_enhancement_emb,
                         intra_linguistic_memory_enhancement_emb])
    sums = _dense_losses(mel_predictions, postnet_mel_predictions,
                         mel_targets, attn_soft[:, 0], attn_hard[:, 0],
                         mel_lens.reshape(B, 1), a_stack, b_stack)
    ctc = _ctc_forward_sum(attn_logprob, src_lens, mel_lens)

    denom = jnp.sum(mel_lens).astype(jnp.float32) * N_MEL
    mel_loss = sums[0] / denom
    postnet_mel_loss = sums[1] / denom
    bw = jnp.where(step < BIN_ENABLE, 0.0,
                   jnp.minimum((step - BIN_ENABLE) / BIN_WARMUP, 1.0))
    bin_loss = -(sums[2] / sums[3]) * bw
    cos_losses = jnp.sum(sums[4:8]) / (B * K_EMB * K_EMB)
    return mel_loss + postnet_mel_loss + ctc + bin_loss + cos_losses
